# baseline (speedup 1.0000x reference)
import functools

import jax
import jax.numpy as jnp
from jax import lax
from jax.experimental import pallas as pl
from jax.experimental.pallas import tpu as pltpu
from jax.experimental.pallas import tpu_sc as plsc

_SC_CORES = 2
_SC_SUBCORES = 16
_SC_WORKERS = _SC_CORES * _SC_SUBCORES
_LANES = 16


def _logits_body(gate_ref, hs_ref, out_ref):
    out_ref[...] = lax.dot_general(
        gate_ref[...], hs_ref[...], (((1,), (1,)), ((), ())),
        preferred_element_type=jnp.float32)


def _sc_scale_body(logits_hbm, out_hbm, lbuf, sbuf, *, tpw, nexp):
    wid = lax.axis_index("s") * _SC_CORES + lax.axis_index("c")
    base = wid * tpw
    pltpu.sync_copy(logits_hbm.at[:, pl.ds(base, tpw)], lbuf)
    for j in range(tpw // _LANES):
        sl = pl.ds(j * _LANES, _LANES)
        vs = [lbuf[e, sl] for e in range(nexp)]
        m1 = vs[0]
        for v in vs[1:]:
            m1 = jnp.maximum(m1, v)
        cnt = jnp.zeros((_LANES,), jnp.float32)
        m2 = jnp.full((_LANES,), -jnp.inf, jnp.float32)
        for v in vs:
            hit = v >= m1
            cnt = cnt + jnp.where(hit, 1.0, 0.0)
            m2 = jnp.maximum(m2, jnp.where(hit, -jnp.inf, v))
        sbuf[sl] = jnp.where(cnt >= 2.0, 2.0 * m1, m1 + m2)
    pltpu.sync_copy(sbuf, out_hbm.at[pl.ds(base, tpw)])


def _fused_body(hs_ref, scale_ref, up_ref, down_ref, ew_ref, g_ref, b_ref,
                out_ref, *, ksteps, kh):
    k = pl.program_id(1)
    hs = hs_ref[...]

    @pl.when(k == 0)
    def _init():
        out_ref[...] = jnp.zeros_like(out_ref)

    u = lax.dot_general(
        hs, up_ref[...], (((1,), (1,)), ((), ())),
        preferred_element_type=jnp.float32)
    a = jnp.maximum(u, 0.0)
    a = (a * a).astype(jnp.bfloat16)
    part = lax.dot_general(
        a, down_ref[...], (((1,), (1,)), ((), ())),
        preferred_element_type=jnp.float32)
    hs_k = hs_ref[:, pl.ds(k * kh, kh)]
    moe = lax.dot_general(
        hs_k, ew_ref[...], (((1,), (0,)), ((), ())),
        preferred_element_type=jnp.float32)
    out_ref[...] += part + moe * scale_ref[...]

    @pl.when(k == ksteps - 1)
    def _finish():
        acc = out_ref[...]
        mu = jnp.mean(acc, axis=-1, keepdims=True)
        var = jnp.mean((acc - mu) ** 2, axis=-1, keepdims=True)
        out_ref[...] = ((acc - mu) * lax.rsqrt(var + 1e-5)
                        * g_ref[...] + b_ref[...])


def kernel(hidden_states, gate_w, up_w, down_w, expert_weight, ln_gamma,
           ln_beta):
    tokens, hidden = hidden_states.shape
    inter = up_w.shape[0]
    nexp = gate_w.shape[0]

    ksteps = 8
    ki = inter // ksteps
    kh = hidden // ksteps
    tm = 1024 if tokens % 1024 == 0 else tokens
    tc = 2048 if tokens % 2048 == 0 else tokens

    hs = hidden_states.astype(jnp.bfloat16)
    gate = gate_w.astype(jnp.bfloat16)
    up = up_w.astype(jnp.bfloat16)
    down = down_w.astype(jnp.bfloat16)
    ew = expert_weight.astype(jnp.bfloat16)
    gamma = ln_gamma.reshape(1, hidden)
    beta = ln_beta.reshape(1, hidden)

    logits_t = pl.pallas_call(
        _logits_body,
        grid=(tokens // tc,),
        in_specs=[
            pl.BlockSpec((nexp, hidden), lambda i: (0, 0)),
            pl.BlockSpec((tc, hidden), lambda i: (i, 0)),
        ],
        out_specs=pl.BlockSpec((nexp, tc), lambda i: (0, i)),
        out_shape=jax.ShapeDtypeStruct((nexp, tokens), jnp.float32),
    )(gate, hs)

    tpw = tokens // _SC_WORKERS
    sc_scale = pl.kernel(
        functools.partial(_sc_scale_body, tpw=tpw, nexp=nexp),
        out_type=jax.ShapeDtypeStruct((tokens,), jnp.float32),
        mesh=plsc.VectorSubcoreMesh(core_axis_name="c", subcore_axis_name="s"),
        scratch_types=[
            pltpu.VMEM((nexp, tpw), jnp.float32),
            pltpu.VMEM((tpw,), jnp.float32),
        ],
    )
    scale = sc_scale(logits_t).reshape(tokens, 1)

    grid = (tokens // tm, ksteps)
    out = pl.pallas_call(
        functools.partial(_fused_body, ksteps=ksteps, kh=kh),
        grid=grid,
        in_specs=[
            pl.BlockSpec((tm, hidden), lambda i, k: (i, 0)),
            pl.BlockSpec((tm, 1), lambda i, k: (i, 0)),
            pl.BlockSpec((ki, hidden), lambda i, k: (k, 0)),
            pl.BlockSpec((hidden, ki), lambda i, k: (0, k)),
            pl.BlockSpec((kh, hidden), lambda i, k: (k, 0)),
            pl.BlockSpec((1, hidden), lambda i, k: (0, 0)),
            pl.BlockSpec((1, hidden), lambda i, k: (0, 0)),
        ],
        out_specs=pl.BlockSpec((tm, hidden), lambda i, k: (i, 0)),
        out_shape=jax.ShapeDtypeStruct((tokens, hidden), jnp.float32),
        compiler_params=pltpu.CompilerParams(
            dimension_semantics=("parallel", "arbitrary")),
    )(hs, scale, up, down, ew, gamma, beta)
    return out


# speedup vs baseline: 1.0190x; 1.0190x over previous
import functools

import jax
import jax.numpy as jnp
from jax import lax
from jax.experimental import pallas as pl
from jax.experimental.pallas import tpu as pltpu
from jax.experimental.pallas import tpu_sc as plsc

_SC_CORES = 2
_SC_SUBCORES = 16
_SC_WORKERS = _SC_CORES * _SC_SUBCORES
_LANES = 16


def _logits_body(gate_ref, hs_ref, out_ref):
    out_ref[...] = lax.dot_general(
        gate_ref[...], hs_ref[...], (((1,), (1,)), ((), ())),
        preferred_element_type=jnp.float32)


def _sc_scale_body(logits_hbm, out_hbm, lbuf, sbuf, *, tpw, nexp):
    wid = lax.axis_index("s") * _SC_CORES + lax.axis_index("c")
    base = wid * tpw
    pltpu.sync_copy(logits_hbm.at[:, pl.ds(base, tpw)], lbuf)
    for j in range(tpw // _LANES):
        sl = pl.ds(j * _LANES, _LANES)
        vs = [lbuf[e, sl] for e in range(nexp)]
        m1 = vs[0]
        for v in vs[1:]:
            m1 = jnp.maximum(m1, v)
        cnt = jnp.zeros((_LANES,), jnp.float32)
        m2 = jnp.full((_LANES,), -jnp.inf, jnp.float32)
        for v in vs:
            hit = v >= m1
            cnt = cnt + jnp.where(hit, 1.0, 0.0)
            m2 = jnp.maximum(m2, jnp.where(hit, -jnp.inf, v))
        sbuf[sl] = jnp.where(cnt >= 2.0, 2.0 * m1, m1 + m2)
    pltpu.sync_copy(sbuf, out_hbm.at[pl.ds(base, tpw)])


def _shared_body(hs_ref, up_ref, down_ref, out_ref, *, ksteps):
    k = pl.program_id(1)
    u = lax.dot_general(
        hs_ref[...], up_ref[...], (((1,), (1,)), ((), ())),
        preferred_element_type=jnp.float32)
    a = jnp.maximum(u, 0.0)
    a = (a * a).astype(jnp.bfloat16)
    part = lax.dot_general(
        a, down_ref[...], (((1,), (1,)), ((), ())),
        preferred_element_type=jnp.float32)
    prev = jnp.where(k == 0, jnp.zeros_like(part), out_ref[...])
    out_ref[...] = part + prev


def _moe_ln_body(hs_ref, scale_ref, acc_ref, ew_ref, gb_ref, out_ref):
    moe = lax.dot_general(
        hs_ref[...], ew_ref[...], (((1,), (0,)), ((), ())),
        preferred_element_type=jnp.float32)
    acc = acc_ref[...] + moe * scale_ref[...]
    mu = jnp.mean(acc, axis=-1, keepdims=True)
    var = jnp.mean((acc - mu) ** 2, axis=-1, keepdims=True)
    out_ref[...] = ((acc - mu) * lax.rsqrt(var + 1e-5)
                    * gb_ref[0:1, :] + gb_ref[1:2, :])


def kernel(hidden_states, gate_w, up_w, down_w, expert_weight, ln_gamma,
           ln_beta):
    tokens, hidden = hidden_states.shape
    inter = up_w.shape[0]
    nexp = gate_w.shape[0]

    ksteps = 8
    ki = inter // ksteps
    tm = 1024 if tokens % 1024 == 0 else tokens
    tn = 512 if tokens % 512 == 0 else tokens
    tc = 2048 if tokens % 2048 == 0 else tokens

    hs = hidden_states.astype(jnp.bfloat16)
    gate = gate_w.astype(jnp.bfloat16)
    up = up_w.astype(jnp.bfloat16)
    down = down_w.astype(jnp.bfloat16)
    ew = expert_weight.astype(jnp.bfloat16)
    gb = jnp.concatenate([ln_gamma.reshape(1, hidden),
                          ln_beta.reshape(1, hidden)], axis=0)

    logits_t = pl.pallas_call(
        _logits_body,
        grid=(tokens // tc,),
        in_specs=[
            pl.BlockSpec((nexp, hidden), lambda i: (0, 0)),
            pl.BlockSpec((tc, hidden), lambda i: (i, 0)),
        ],
        out_specs=pl.BlockSpec((nexp, tc), lambda i: (0, i)),
        out_shape=jax.ShapeDtypeStruct((nexp, tokens), jnp.float32),
    )(gate, hs)

    tpw = tokens // _SC_WORKERS
    sc_scale = pl.kernel(
        functools.partial(_sc_scale_body, tpw=tpw, nexp=nexp),
        out_type=jax.ShapeDtypeStruct((tokens,), jnp.float32),
        mesh=plsc.VectorSubcoreMesh(core_axis_name="c", subcore_axis_name="s"),
        scratch_types=[
            pltpu.VMEM((nexp, tpw), jnp.float32),
            pltpu.VMEM((tpw,), jnp.float32),
        ],
    )
    scale = sc_scale(logits_t).reshape(tokens, 1)

    acc = pl.pallas_call(
        functools.partial(_shared_body, ksteps=ksteps),
        grid=(tokens // tm, ksteps),
        in_specs=[
            pl.BlockSpec((tm, hidden), lambda i, k: (i, 0)),
            pl.BlockSpec((ki, hidden), lambda i, k: (k, 0)),
            pl.BlockSpec((hidden, ki), lambda i, k: (0, k)),
        ],
        out_specs=pl.BlockSpec((tm, hidden), lambda i, k: (i, 0)),
        out_shape=jax.ShapeDtypeStruct((tokens, hidden), jnp.float32),
        compiler_params=pltpu.CompilerParams(
            dimension_semantics=("parallel", "arbitrary")),
    )(hs, up, down)

    out = pl.pallas_call(
        _moe_ln_body,
        grid=(tokens // tn,),
        in_specs=[
            pl.BlockSpec((tn, hidden), lambda i: (i, 0)),
            pl.BlockSpec((tn, 1), lambda i: (i, 0)),
            pl.BlockSpec((tn, hidden), lambda i: (i, 0)),
            pl.BlockSpec((hidden, hidden), lambda i: (0, 0)),
            pl.BlockSpec((2, hidden), lambda i: (0, 0)),
        ],
        out_specs=pl.BlockSpec((tn, hidden), lambda i: (i, 0)),
        out_shape=jax.ShapeDtypeStruct((tokens, hidden), jnp.float32),
        compiler_params=pltpu.CompilerParams(
            dimension_semantics=("parallel",)),
    )(hs, scale, acc, ew, gb)
    return out


# speedup vs baseline: 1.0230x; 1.0039x over previous
import functools

import jax
import jax.numpy as jnp
from jax import lax
from jax.experimental import pallas as pl
from jax.experimental.pallas import tpu as pltpu
from jax.experimental.pallas import tpu_sc as plsc

_SC_CORES = 2
_SC_SUBCORES = 16
_SC_WORKERS = _SC_CORES * _SC_SUBCORES
_LANES = 16


def _sc_scale_body(logits_hbm, out_hbm, lbuf, sbuf, *, tpw, nexp):
    wid = lax.axis_index("s") * _SC_CORES + lax.axis_index("c")
    base = wid * tpw
    pltpu.sync_copy(logits_hbm.at[:, pl.ds(base, tpw)], lbuf)
    for j in range(tpw // _LANES):
        sl = pl.ds(j * _LANES, _LANES)
        vs = [lbuf[e, sl] for e in range(nexp)]
        m1 = vs[0]
        for v in vs[1:]:
            m1 = jnp.maximum(m1, v)
        cnt = jnp.zeros((_LANES,), jnp.float32)
        m2 = jnp.full((_LANES,), -jnp.inf, jnp.float32)
        for v in vs:
            hit = v >= m1
            cnt = cnt + jnp.where(hit, 1.0, 0.0)
            m2 = jnp.maximum(m2, jnp.where(hit, -jnp.inf, v))
        sbuf[sl] = jnp.where(cnt >= 2.0, 2.0 * m1, m1 + m2)
    pltpu.sync_copy(sbuf, out_hbm.at[pl.ds(base, tpw)])


def _shared_part(hs, up_ref, down_ref, jsteps, jc):
    part = None
    for j in range(jsteps):
        u = lax.dot_general(
            hs, up_ref[pl.ds(j * jc, jc), :], (((1,), (1,)), ((), ())),
            preferred_element_type=jnp.float32)
        aj = jnp.maximum(u, 0.0)
        aj = (aj * aj).astype(jnp.bfloat16)
        pj = lax.dot_general(
            aj, down_ref[:, pl.ds(j * jc, jc)], (((1,), (1,)), ((), ())),
            preferred_element_type=jnp.float32)
        part = pj if part is None else part + pj
    return part


def _half_a_body(hs_ref, gate_ref, up_ref, down_ref, out_ref, lg_ref, *,
                 jsteps, jc):
    hs = hs_ref[...]
    lg_ref[...] = lax.dot_general(
        gate_ref[...], hs, (((1,), (1,)), ((), ())),
        preferred_element_type=jnp.float32)
    out_ref[...] = _shared_part(hs, up_ref, down_ref, jsteps, jc)


def _half_b_body(hs_ref, scale_ref, acc_ref, up_ref, down_ref, ew_ref,
                 gb_ref, out_ref, *, jsteps, jc):
    hs = hs_ref[...]
    part = _shared_part(hs, up_ref, down_ref, jsteps, jc)
    moe = lax.dot_general(
        hs, ew_ref[...], (((1,), (0,)), ((), ())),
        preferred_element_type=jnp.float32)
    acc = acc_ref[...] + part + moe * scale_ref[...]
    mu = jnp.mean(acc, axis=-1, keepdims=True)
    var = jnp.mean((acc - mu) ** 2, axis=-1, keepdims=True)
    out_ref[...] = ((acc - mu) * lax.rsqrt(var + 1e-5)
                    * gb_ref[0:1, :] + gb_ref[1:2, :])


def kernel(hidden_states, gate_w, up_w, down_w, expert_weight, ln_gamma,
           ln_beta):
    tokens, hidden = hidden_states.shape
    inter = up_w.shape[0]
    nexp = gate_w.shape[0]

    ih = inter // 2
    jc = min(1024, ih)
    jsteps = ih // jc
    ta = 512 if tokens % 512 == 0 else tokens
    tb = 256 if tokens % 256 == 0 else tokens

    hs = hidden_states.astype(jnp.bfloat16)
    gate = gate_w.astype(jnp.bfloat16)
    up = up_w.astype(jnp.bfloat16)
    down = down_w.astype(jnp.bfloat16)
    ew = expert_weight.astype(jnp.bfloat16)
    gb = jnp.concatenate([ln_gamma.reshape(1, hidden),
                          ln_beta.reshape(1, hidden)], axis=0)

    acc, logits_t = pl.pallas_call(
        functools.partial(_half_a_body, jsteps=jsteps, jc=jc),
        grid=(tokens // ta,),
        in_specs=[
            pl.BlockSpec((ta, hidden), lambda i: (i, 0)),
            pl.BlockSpec((nexp, hidden), lambda i: (0, 0)),
            pl.BlockSpec((ih, hidden), lambda i: (0, 0)),
            pl.BlockSpec((hidden, ih), lambda i: (0, 0)),
        ],
        out_specs=[
            pl.BlockSpec((ta, hidden), lambda i: (i, 0)),
            pl.BlockSpec((nexp, ta), lambda i: (0, i)),
        ],
        out_shape=[
            jax.ShapeDtypeStruct((tokens, hidden), jnp.float32),
            jax.ShapeDtypeStruct((nexp, tokens), jnp.float32),
        ],
        compiler_params=pltpu.CompilerParams(
            dimension_semantics=("parallel",)),
    )(hs, gate, up, down)

    tpw = tokens // _SC_WORKERS
    sc_scale = pl.kernel(
        functools.partial(_sc_scale_body, tpw=tpw, nexp=nexp),
        out_type=jax.ShapeDtypeStruct((tokens,), jnp.float32),
        mesh=plsc.VectorSubcoreMesh(core_axis_name="c", subcore_axis_name="s"),
        scratch_types=[
            pltpu.VMEM((nexp, tpw), jnp.float32),
            pltpu.VMEM((tpw,), jnp.float32),
        ],
    )
    scale = sc_scale(logits_t).reshape(tokens, 1)

    out = pl.pallas_call(
        functools.partial(_half_b_body, jsteps=jsteps, jc=jc),
        grid=(tokens // tb,),
        in_specs=[
            pl.BlockSpec((tb, hidden), lambda i: (i, 0)),
            pl.BlockSpec((tb, 1), lambda i: (i, 0)),
            pl.BlockSpec((tb, hidden), lambda i: (i, 0)),
            pl.BlockSpec((ih, hidden), lambda i: (1, 0)),
            pl.BlockSpec((hidden, ih), lambda i: (0, 1)),
            pl.BlockSpec((hidden, hidden), lambda i: (0, 0)),
            pl.BlockSpec((2, hidden), lambda i: (0, 0)),
        ],
        out_specs=pl.BlockSpec((tb, hidden), lambda i: (i, 0)),
        out_shape=jax.ShapeDtypeStruct((tokens, hidden), jnp.float32),
        compiler_params=pltpu.CompilerParams(
            dimension_semantics=("parallel",)),
    )(hs, scale, acc, up, down, ew, gb)
    return out


# speedup vs baseline: 1.0647x; 1.0408x over previous
import functools

import jax
import jax.numpy as jnp
from jax import lax
from jax.experimental import pallas as pl
from jax.experimental.pallas import tpu as pltpu
from jax.experimental.pallas import tpu_sc as plsc

_SC_CORES = 2
_SC_SUBCORES = 16
_SC_WORKERS = _SC_CORES * _SC_SUBCORES
_LANES = 16


def _sc_scale_body(logits_hbm, out_hbm, lbuf, sbuf, *, tpw, nexp):
    wid = lax.axis_index("s") * _SC_CORES + lax.axis_index("c")
    base = wid * tpw
    pltpu.sync_copy(logits_hbm.at[:, pl.ds(base, tpw)], lbuf)
    for j in range(tpw // _LANES):
        sl = pl.ds(j * _LANES, _LANES)
        vs = [lbuf[e, sl] for e in range(nexp)]
        m1 = vs[0]
        for v in vs[1:]:
            m1 = jnp.maximum(m1, v)
        cnt = jnp.zeros((_LANES,), jnp.float32)
        m2 = jnp.full((_LANES,), -jnp.inf, jnp.float32)
        for v in vs:
            hit = v >= m1
            cnt = cnt + jnp.where(hit, 1.0, 0.0)
            m2 = jnp.maximum(m2, jnp.where(hit, -jnp.inf, v))
        sbuf[sl] = jnp.where(cnt >= 2.0, 2.0 * m1, m1 + m2)
    pltpu.sync_copy(sbuf, out_hbm.at[pl.ds(base, tpw)])


def _shared_part(hs, up_ref, down_ref, jsteps, jc):
    part = None
    for j in range(jsteps):
        u = lax.dot_general(
            hs, up_ref[pl.ds(j * jc, jc), :], (((1,), (1,)), ((), ())),
            preferred_element_type=jnp.float32)
        aj = jnp.maximum(u, 0.0)
        aj = (aj * aj).astype(jnp.bfloat16)
        pj = lax.dot_general(
            aj, down_ref[:, pl.ds(j * jc, jc)], (((1,), (1,)), ((), ())),
            preferred_element_type=jnp.float32)
        part = pj if part is None else part + pj
    return part


def _half_a_body(hs_ref, gate_ref, up_ref, down_ref, out_ref, lg_ref, *,
                 jsteps, jc):
    hs = hs_ref[...].astype(jnp.bfloat16)
    lg_ref[...] = lax.dot_general(
        gate_ref[...], hs, (((1,), (1,)), ((), ())),
        preferred_element_type=jnp.float32)
    out_ref[...] = _shared_part(hs, up_ref, down_ref, jsteps, jc)


def _half_b_body(hs_ref, scale_ref, acc_ref, up_ref, down_ref, ew_ref,
                 gb_ref, out_ref, *, jsteps, jc):
    hs = hs_ref[...].astype(jnp.bfloat16)
    part = _shared_part(hs, up_ref, down_ref, jsteps, jc)
    moe = lax.dot_general(
        hs, ew_ref[...], (((1,), (0,)), ((), ())),
        preferred_element_type=jnp.float32)
    acc = acc_ref[...] + part + moe * scale_ref[...]
    mu = jnp.mean(acc, axis=-1, keepdims=True)
    var = jnp.mean((acc - mu) ** 2, axis=-1, keepdims=True)
    out_ref[...] = ((acc - mu) * lax.rsqrt(var + 1e-5)
                    * gb_ref[0:1, :] + gb_ref[1:2, :])


def kernel(hidden_states, gate_w, up_w, down_w, expert_weight, ln_gamma,
           ln_beta):
    tokens, hidden = hidden_states.shape
    inter = up_w.shape[0]
    nexp = gate_w.shape[0]

    ih = inter // 2
    jc = min(1024, ih)
    jsteps = ih // jc
    ta = 512 if tokens % 512 == 0 else tokens
    tb = 256 if tokens % 256 == 0 else tokens

    hs = hidden_states
    gate = gate_w.astype(jnp.bfloat16)
    up = up_w.astype(jnp.bfloat16)
    down = down_w.astype(jnp.bfloat16)
    ew = expert_weight.astype(jnp.bfloat16)
    gb = jnp.concatenate([ln_gamma.reshape(1, hidden),
                          ln_beta.reshape(1, hidden)], axis=0)

    acc, logits_t = pl.pallas_call(
        functools.partial(_half_a_body, jsteps=jsteps, jc=jc),
        grid=(tokens // ta,),
        in_specs=[
            pl.BlockSpec((ta, hidden), lambda i: (i, 0)),
            pl.BlockSpec((nexp, hidden), lambda i: (0, 0)),
            pl.BlockSpec((ih, hidden), lambda i: (0, 0)),
            pl.BlockSpec((hidden, ih), lambda i: (0, 0)),
        ],
        out_specs=[
            pl.BlockSpec((ta, hidden), lambda i: (i, 0)),
            pl.BlockSpec((nexp, ta), lambda i: (0, i)),
        ],
        out_shape=[
            jax.ShapeDtypeStruct((tokens, hidden), jnp.float32),
            jax.ShapeDtypeStruct((nexp, tokens), jnp.float32),
        ],
        compiler_params=pltpu.CompilerParams(
            dimension_semantics=("parallel",)),
    )(hs, gate, up, down)

    tpw = tokens // _SC_WORKERS
    sc_scale = pl.kernel(
        functools.partial(_sc_scale_body, tpw=tpw, nexp=nexp),
        out_type=jax.ShapeDtypeStruct((tokens,), jnp.float32),
        mesh=plsc.VectorSubcoreMesh(core_axis_name="c", subcore_axis_name="s"),
        scratch_types=[
            pltpu.VMEM((nexp, tpw), jnp.float32),
            pltpu.VMEM((tpw,), jnp.float32),
        ],
    )
    scale = sc_scale(logits_t).reshape(tokens, 1)

    out = pl.pallas_call(
        functools.partial(_half_b_body, jsteps=jsteps, jc=jc),
        grid=(tokens // tb,),
        in_specs=[
            pl.BlockSpec((tb, hidden), lambda i: (i, 0)),
            pl.BlockSpec((tb, 1), lambda i: (i, 0)),
            pl.BlockSpec((tb, hidden), lambda i: (i, 0)),
            pl.BlockSpec((ih, hidden), lambda i: (1, 0)),
            pl.BlockSpec((hidden, ih), lambda i: (0, 1)),
            pl.BlockSpec((hidden, hidden), lambda i: (0, 0)),
            pl.BlockSpec((2, hidden), lambda i: (0, 0)),
        ],
        out_specs=pl.BlockSpec((tb, hidden), lambda i: (i, 0)),
        out_shape=jax.ShapeDtypeStruct((tokens, hidden), jnp.float32),
        compiler_params=pltpu.CompilerParams(
            dimension_semantics=("parallel",)),
    )(hs, scale, acc, up, down, ew, gb)
    return out
